# Initial kernel scaffold; baseline (speedup 1.0000x reference)
import jax
import jax.numpy as jnp
from jax.experimental import pallas as pl


def kernel(input):
    raise NotImplementedError("write your pallas kernel here")


# speedup vs baseline: 8.3968x; 8.3968x over previous
import jax
import jax.numpy as jnp
from jax import lax
from jax.experimental import pallas as pl
from jax.experimental.pallas import tpu as pltpu

_B, _H, _W = 4, 384, 384
_K = 64
_IBIG = 2**31 - 1


def _bbox_kernel(x_ref, out_ref):
    tm = x_ref[0, 0]
    lm = x_ref[0, 1]
    mask = (tm > 0.4) | (lm > 0.4)

    row = lax.broadcasted_iota(jnp.int32, (_H, _W), 0)
    col = lax.broadcasted_iota(jnp.int32, (_H, _W), 1)
    lin = row * _W + col + 1

    lab0 = jnp.where(mask, lin, _IBIG)

    ibig_row = jnp.full((1, _W), _IBIG, jnp.int32)
    ibig_col = jnp.full((_H, 1), _IBIG, jnp.int32)

    def prop(lab):
        up = jnp.concatenate([lab[1:, :], ibig_row], axis=0)
        dn = jnp.concatenate([ibig_row, lab[:-1, :]], axis=0)
        lf = jnp.concatenate([lab[:, 1:], ibig_col], axis=1)
        rt = jnp.concatenate([ibig_col, lab[:, :-1]], axis=1)
        n = jnp.minimum(jnp.minimum(up, dn), jnp.minimum(lf, rt))
        return jnp.where(mask, jnp.minimum(lab, n), _IBIG)

    def cond(st):
        return st[1]

    def body(st):
        lab, _ = st
        new = prop(prop(lab))
        return new, jnp.any(new != lab)

    lab, _ = lax.while_loop(cond, body, (lab0, jnp.bool_(True)))

    rootlab0 = jnp.where(lab == lin, lab, _IBIG)
    labm = jnp.where(mask, lab, 0)

    col_idx = lax.broadcasted_iota(jnp.int32, (1, _W), 1)
    row_idx = lax.broadcasted_iota(jnp.int32, (_H, 1), 0)
    k_idx = lax.broadcasted_iota(jnp.int32, (1, _K), 1)
    z_lane = jnp.zeros((1, 2), jnp.int32)
    z_sub = jnp.zeros((2, 1), jnp.int32)

    def sh3_lane(v):
        l = jnp.concatenate([v[:, 2:], z_lane], axis=1)
        r = jnp.concatenate([z_lane, v[:, :-2]], axis=1)
        return v + l + r

    def sh3_sub(v):
        u = jnp.concatenate([v[2:, :], z_sub], axis=0)
        d = jnp.concatenate([z_sub, v[:-2, :]], axis=0)
        return v + u + d

    def kbody(k, carry):
        rootlab, ax2, ay2, aw, ah = carry
        idk = jnp.min(rootlab)
        rootlab = jnp.where(rootlab == idk, _IBIG, rootlab)

        eq = labm == idk
        eqi = eq.astype(jnp.int32)
        colS = jnp.max(eqi, axis=0, keepdims=True)
        rowS = jnp.max(eqi, axis=1, keepdims=True)
        colTop = eqi[0:1, :] + eqi[2:3, :]
        rowLeft = eqi[:, 0:1] + eqi[:, 2:3]

        m0 = (sh3_lane(colS) > 0) & (sh3_lane(colTop) == 0)
        m1 = (sh3_sub(rowS) > 0) & (sh3_sub(rowLeft) == 0)

        mn0 = jnp.min(jnp.where(m0, col_idx, _W))
        mx0 = jnp.max(jnp.where(m0, col_idx, -1))
        any0 = mx0 >= 0
        y2 = jnp.where(any0, mn0, 0)
        h = jnp.where(any0, mx0 - mn0, 1)

        mn1 = jnp.min(jnp.where(m1, row_idx, _H))
        mx1 = jnp.max(jnp.where(m1, row_idx, -1))
        any1 = mx1 >= 0
        x2 = jnp.where(any1, mn1, 0)
        w = jnp.where(any1, mx1 - mn1, 1)

        sel = k_idx == k
        ax2 = jnp.where(sel, x2, ax2)
        ay2 = jnp.where(sel, y2, ay2)
        aw = jnp.where(sel, w, aw)
        ah = jnp.where(sel, h, ah)
        return rootlab, ax2, ay2, aw, ah

    zk = jnp.zeros((1, _K), jnp.int32)
    _, ax2, ay2, aw, ah = lax.fori_loop(0, _K, kbody,
                                        (rootlab0, zk, zk, zk, zk))
    out_ref[0] = jnp.concatenate([ax2, ay2, aw, ah], axis=0)


@jax.jit
def kernel(input):
    xt = jnp.transpose(input, (0, 3, 1, 2))
    out = pl.pallas_call(
        _bbox_kernel,
        grid=(_B,),
        in_specs=[pl.BlockSpec((1, 2, _H, _W), lambda b: (b, 0, 0, 0))],
        out_specs=pl.BlockSpec((1, 4, _K), lambda b: (b, 0, 0)),
        out_shape=jax.ShapeDtypeStruct((_B, 4, _K), jnp.int32),
        compiler_params=pltpu.CompilerParams(
            dimension_semantics=("parallel",)),
    )(xt)
    return jnp.transpose(out, (0, 2, 1))


# speedup vs baseline: 11.2908x; 1.3446x over previous
import jax
import jax.numpy as jnp
from jax import lax
from jax.experimental import pallas as pl
from jax.experimental.pallas import tpu as pltpu

_B, _H, _W = 4, 384, 384
_K = 64
_IBIG = 2**31 - 1


def _bbox_kernel(x_ref, out_ref):
    tm = x_ref[0, 0]
    lm = x_ref[0, 1]
    mask = (tm > 0.4) | (lm > 0.4)

    row = lax.broadcasted_iota(jnp.int32, (_H, _W), 0)
    col = lax.broadcasted_iota(jnp.int32, (_H, _W), 1)
    lin = row * _W + col + 1

    lab0 = jnp.where(mask, lin, _IBIG)

    maski = mask.astype(jnp.int32)

    def scan_dir(v, shift_v, shift_c):
        c = maski
        for d in (1, 2, 4, 8, 16, 32):
            vs = shift_v(v, d)
            cs = shift_c(c, d)
            v = jnp.minimum(v, jnp.where(c != 0, vs, _IBIG))
            c = c & cs
        return v

    def shr_v(v, d):
        return jnp.concatenate(
            [jnp.full((_H, d), _IBIG, jnp.int32), v[:, :-d]], axis=1)

    def shr_c(c, d):
        return jnp.concatenate(
            [jnp.zeros((_H, d), jnp.int32), c[:, :-d]], axis=1)

    def shl_v(v, d):
        return jnp.concatenate(
            [v[:, d:], jnp.full((_H, d), _IBIG, jnp.int32)], axis=1)

    def shl_c(c, d):
        return jnp.concatenate(
            [c[:, d:], jnp.zeros((_H, d), jnp.int32)], axis=1)

    def shd_v(v, d):
        return jnp.concatenate(
            [jnp.full((d, _W), _IBIG, jnp.int32), v[:-d, :]], axis=0)

    def shd_c(c, d):
        return jnp.concatenate(
            [jnp.zeros((d, _W), jnp.int32), c[:-d, :]], axis=0)

    def shu_v(v, d):
        return jnp.concatenate(
            [v[d:, :], jnp.full((d, _W), _IBIG, jnp.int32)], axis=0)

    def shu_c(c, d):
        return jnp.concatenate(
            [c[d:, :], jnp.zeros((d, _W), jnp.int32)], axis=0)

    def sweep(v):
        v = scan_dir(v, shd_v, shd_c)
        v = scan_dir(v, shu_v, shu_c)
        v = scan_dir(v, shr_v, shr_c)
        v = scan_dir(v, shl_v, shl_c)
        return v

    def cond(st):
        return st[1]

    def body(st):
        lab, _ = st
        new = sweep(lab)
        return new, jnp.any(new != lab)

    lab, _ = lax.while_loop(cond, body, (sweep(lab0), jnp.bool_(True)))

    rootlab0 = jnp.where(lab == lin, lab, _IBIG)
    labm = jnp.where(mask, lab, 0)

    col_idx = lax.broadcasted_iota(jnp.int32, (1, _W), 1)
    row_idx = lax.broadcasted_iota(jnp.int32, (_H, 1), 0)
    k_idx = lax.broadcasted_iota(jnp.int32, (1, _K), 1)
    z_lane = jnp.zeros((1, 2), jnp.int32)
    z_sub = jnp.zeros((2, 1), jnp.int32)

    def sh3_lane(v):
        l = jnp.concatenate([v[:, 2:], z_lane], axis=1)
        r = jnp.concatenate([z_lane, v[:, :-2]], axis=1)
        return v + l + r

    def sh3_sub(v):
        u = jnp.concatenate([v[2:, :], z_sub], axis=0)
        d = jnp.concatenate([z_sub, v[:-2, :]], axis=0)
        return v + u + d

    def kbody(k, carry):
        rootlab, ax2, ay2, aw, ah = carry
        idk = jnp.min(rootlab)
        rootlab = jnp.where(rootlab == idk, _IBIG, rootlab)

        eq = labm == idk
        eqi = eq.astype(jnp.int32)
        colS = jnp.max(eqi, axis=0, keepdims=True)
        rowS = jnp.max(eqi, axis=1, keepdims=True)
        colTop = eqi[0:1, :] + eqi[2:3, :]
        rowLeft = eqi[:, 0:1] + eqi[:, 2:3]

        m0 = (sh3_lane(colS) > 0) & (sh3_lane(colTop) == 0)
        m1 = (sh3_sub(rowS) > 0) & (sh3_sub(rowLeft) == 0)

        mn0 = jnp.min(jnp.where(m0, col_idx, _W))
        mx0 = jnp.max(jnp.where(m0, col_idx, -1))
        any0 = mx0 >= 0
        y2 = jnp.where(any0, mn0, 0)
        h = jnp.where(any0, mx0 - mn0, 1)

        mn1 = jnp.min(jnp.where(m1, row_idx, _H))
        mx1 = jnp.max(jnp.where(m1, row_idx, -1))
        any1 = mx1 >= 0
        x2 = jnp.where(any1, mn1, 0)
        w = jnp.where(any1, mx1 - mn1, 1)

        sel = k_idx == k
        ax2 = jnp.where(sel, x2, ax2)
        ay2 = jnp.where(sel, y2, ay2)
        aw = jnp.where(sel, w, aw)
        ah = jnp.where(sel, h, ah)
        return rootlab, ax2, ay2, aw, ah

    zk = jnp.zeros((1, _K), jnp.int32)
    _, ax2, ay2, aw, ah = lax.fori_loop(0, _K, kbody,
                                        (rootlab0, zk, zk, zk, zk))
    out_ref[0] = jnp.concatenate([ax2, ay2, aw, ah], axis=0)


@jax.jit
def kernel(input):
    xt = jnp.transpose(input, (0, 3, 1, 2))
    out = pl.pallas_call(
        _bbox_kernel,
        grid=(_B,),
        in_specs=[pl.BlockSpec((1, 2, _H, _W), lambda b: (b, 0, 0, 0))],
        out_specs=pl.BlockSpec((1, 4, _K), lambda b: (b, 0, 0)),
        out_shape=jax.ShapeDtypeStruct((_B, 4, _K), jnp.int32),
        compiler_params=pltpu.CompilerParams(
            dimension_semantics=("parallel",)),
    )(xt)
    return jnp.transpose(out, (0, 2, 1))
